# baseline (speedup 1.0000x reference)
import functools

import jax
import jax.numpy as jnp
from jax import lax
from jax.experimental import pallas as pl
from jax.experimental.pallas import tpu as pltpu
from jax.experimental.pallas import tpu_sc as plsc

N = 10000
E = 320000
D = 128
EPS = 1e-5

NUM_CORES = 2
NUM_SUBCORES = 16
NUM_WORKERS = NUM_CORES * NUM_SUBCORES
CHUNK = 128
NCHUNKS = E // CHUNK
BASE_CHUNKS = NCHUNKS // NUM_WORKERS
EXTRA_TILES = NCHUNKS - BASE_CHUNKS * NUM_WORKERS
PHASE_A = 40
PHASE_B_MAX = BASE_CHUNKS + 1 - PHASE_A
STRIPE = 624
LAST_STRIPE = N - 15 * STRIPE
ZROWS = 16


def _sc_segment_sum(eidx, h):
    mesh = plsc.VectorSubcoreMesh(
        core_axis_name="c", subcore_axis_name="s")

    @functools.partial(
        pl.kernel,
        out_type=jax.ShapeDtypeStruct((NUM_CORES, N, D), jnp.float32),
        mesh=mesh,
        compiler_params=pltpu.CompilerParams(use_tc_tiling_on_sc=False),
        scratch_types=[
            pltpu.VMEM_SHARED((N, D), jnp.float32),
            pltpu.VMEM((PHASE_A * CHUNK,), jnp.int32),
            pltpu.VMEM((PHASE_A * CHUNK,), jnp.int32),
            pltpu.VMEM((CHUNK, D), jnp.float32),
            pltpu.VMEM((CHUNK, D), jnp.float32),
            pltpu.VMEM((ZROWS, D), jnp.float32),
            pltpu.SemaphoreType.DMA,
            pltpu.SemaphoreType.DMA,
        ],
    )
    def seg_sum(eidx_hbm, h_hbm, out_hbm,
                accum, src_v, dst_v, buf0, buf1, zbuf, sem0, sem1):
        cid = lax.axis_index("c")
        sid = lax.axis_index("s")
        wid = cid * NUM_SUBCORES + sid
        start = wid * BASE_CHUNKS + jnp.minimum(wid, EXTRA_TILES)
        has_extra = wid < EXTRA_TILES

        zvec = jnp.zeros((16,), jnp.float32)

        def zfill(i, carry):
            for l in range(D // 16):
                zbuf[i, pl.ds(l * 16, 16)] = zvec
            return carry

        lax.fori_loop(0, ZROWS, zfill, 0)
        for z in range(LAST_STRIPE // ZROWS):
            pltpu.async_copy(
                zbuf, accum.at[pl.ds(sid * STRIPE + z * ZROWS, ZROWS)], sem0)
        for z in range(LAST_STRIPE // ZROWS):
            pltpu.make_async_copy(
                zbuf, accum.at[pl.ds(sid * STRIPE + z * ZROWS, ZROWS)],
                sem0).wait()

        plsc.subcore_barrier()

        def src_at(j):
            return src_v.at[pl.ds(j * CHUNK, CHUNK)]

        def dst_at(j):
            return dst_v.at[pl.ds(j * CHUNK, CHUNK)]

        def make_body(nb):
            def body(i, carry):
                j0 = 2 * i
                pltpu.make_async_copy(
                    h_hbm.at[src_at(j0)], buf0, sem0).wait()
                pltpu.sync_copy(buf0, accum.at[dst_at(j0)], add=True)

                @pl.when(j0 + 2 < nb)
                def _():
                    pltpu.async_copy(h_hbm.at[src_at(j0 + 2)], buf0, sem0)

                j1 = j0 + 1
                pltpu.make_async_copy(
                    h_hbm.at[src_at(j1)], buf1, sem1).wait()
                pltpu.sync_copy(buf1, accum.at[dst_at(j1)], add=True)

                @pl.when(j1 + 2 < nb)
                def _():
                    pltpu.async_copy(h_hbm.at[src_at(j1 + 2)], buf1, sem1)

                return carry
            return body

        ea = start * CHUNK
        pltpu.sync_copy(eidx_hbm.at[0, pl.ds(ea, PHASE_A * CHUNK)], src_v)
        pltpu.sync_copy(eidx_hbm.at[1, pl.ds(ea, PHASE_A * CHUNK)], dst_v)
        pltpu.async_copy(h_hbm.at[src_at(0)], buf0, sem0)
        pltpu.async_copy(h_hbm.at[src_at(1)], buf1, sem1)
        lax.fori_loop(0, PHASE_A // 2, make_body(PHASE_A), 0)

        eb = (start + PHASE_A) * CHUNK
        nb_base = (PHASE_B_MAX - 1) * CHUNK
        pltpu.sync_copy(eidx_hbm.at[0, pl.ds(eb, nb_base)],
                        src_v.at[pl.ds(0, nb_base)])
        pltpu.sync_copy(eidx_hbm.at[1, pl.ds(eb, nb_base)],
                        dst_v.at[pl.ds(0, nb_base)])

        @pl.when(has_extra)
        def _():
            pltpu.sync_copy(eidx_hbm.at[0, pl.ds(eb + nb_base, CHUNK)],
                            src_v.at[pl.ds(nb_base, CHUNK)])
            pltpu.sync_copy(eidx_hbm.at[1, pl.ds(eb + nb_base, CHUNK)],
                            dst_v.at[pl.ds(nb_base, CHUNK)])

        nb = jnp.where(has_extra, PHASE_B_MAX, PHASE_B_MAX - 1)
        pltpu.async_copy(h_hbm.at[src_at(0)], buf0, sem0)
        pltpu.async_copy(h_hbm.at[src_at(1)], buf1, sem1)
        lax.fori_loop(0, (PHASE_B_MAX - 1) // 2, make_body(nb), 0)

        @pl.when(has_extra)
        def _():
            j = PHASE_B_MAX - 1
            pltpu.make_async_copy(h_hbm.at[src_at(j)], buf0, sem0).wait()
            pltpu.sync_copy(buf0, accum.at[dst_at(j)], add=True)

        plsc.subcore_barrier()

        pltpu.sync_copy(
            accum.at[pl.ds(sid * STRIPE, STRIPE)],
            out_hbm.at[cid, pl.ds(sid * STRIPE, STRIPE)])

        @pl.when(sid == NUM_SUBCORES - 1)
        def _():
            pltpu.sync_copy(
                accum.at[pl.ds(15 * STRIPE + STRIPE, LAST_STRIPE - STRIPE)],
                out_hbm.at[cid, pl.ds(15 * STRIPE + STRIPE,
                                      LAST_STRIPE - STRIPE)])

    return seg_sum(eidx, h)


def _bn_relu_body(part_ref, h_in_ref, gamma_ref, beta_ref, out_ref):
    x = part_ref[0] + part_ref[1] + h_in_ref[...]
    mean = jnp.mean(x, axis=0, keepdims=True)
    xc = x - mean
    var = jnp.mean(xc * xc, axis=0, keepdims=True)
    inv = lax.rsqrt(var + EPS)
    y = xc * inv * gamma_ref[...] + beta_ref[...]
    out_ref[...] = jnp.maximum(y, 0.0)


def _tc_bn_relu(partials, h_in, gamma, beta):
    return pl.pallas_call(
        _bn_relu_body,
        out_shape=jax.ShapeDtypeStruct((N, D), jnp.float32),
    )(partials, h_in, gamma.reshape(1, D), beta.reshape(1, D))


def kernel(edge_index, h, h_in, gamma, beta):
    partials = _sc_segment_sum(edge_index, h)
    return _tc_bn_relu(partials, h_in, gamma, beta)


# speedup vs baseline: 1.0023x; 1.0023x over previous
import functools

import jax
import jax.numpy as jnp
from jax import lax
from jax.experimental import pallas as pl
from jax.experimental.pallas import tpu as pltpu
from jax.experimental.pallas import tpu_sc as plsc

N = 10000
E = 320000
D = 128
EPS = 1e-5

NUM_CORES = 2
NUM_SUBCORES = 16
NUM_WORKERS = NUM_CORES * NUM_SUBCORES
CHUNK = 128
NCHUNKS = E // CHUNK
BASE_CHUNKS = NCHUNKS // NUM_WORKERS
EXTRA_TILES = NCHUNKS - BASE_CHUNKS * NUM_WORKERS
PHASE_A = 40
PHASE_B_MAX = BASE_CHUNKS + 1 - PHASE_A
STRIPE = 624
LAST_STRIPE = N - 15 * STRIPE
ZROWS = 16


def _sc_segment_sum(eidx, h):
    mesh = plsc.VectorSubcoreMesh(
        core_axis_name="c", subcore_axis_name="s")

    @functools.partial(
        pl.kernel,
        out_type=jax.ShapeDtypeStruct((NUM_CORES, N, D), jnp.float32),
        mesh=mesh,
        compiler_params=pltpu.CompilerParams(use_tc_tiling_on_sc=False),
        scratch_types=[
            pltpu.VMEM_SHARED((N, D), jnp.float32),
            pltpu.VMEM((PHASE_A, CHUNK), jnp.int32),
            pltpu.VMEM((PHASE_A, CHUNK), jnp.int32),
            pltpu.VMEM((CHUNK, D), jnp.float32),
            pltpu.VMEM((CHUNK, D), jnp.float32),
            pltpu.VMEM((ZROWS, D), jnp.float32),
            pltpu.SemaphoreType.DMA,
            pltpu.SemaphoreType.DMA,
        ],
    )
    def seg_sum(eidx_hbm, h_hbm, out_hbm,
                accum, src_v, dst_v, buf0, buf1, zbuf, sem0, sem1):
        cid = lax.axis_index("c")
        sid = lax.axis_index("s")
        wid = cid * NUM_SUBCORES + sid
        start = wid * BASE_CHUNKS + jnp.minimum(wid, EXTRA_TILES)
        has_extra = wid < EXTRA_TILES

        zvec = jnp.zeros((16,), jnp.float32)

        def zfill(i, carry):
            for l in range(D // 16):
                zbuf[i, pl.ds(l * 16, 16)] = zvec
            return carry

        lax.fori_loop(0, ZROWS, zfill, 0)
        for z in range(LAST_STRIPE // ZROWS):
            pltpu.async_copy(
                zbuf, accum.at[pl.ds(sid * STRIPE + z * ZROWS, ZROWS)], sem0)
        for z in range(LAST_STRIPE // ZROWS):
            pltpu.make_async_copy(
                zbuf, accum.at[pl.ds(sid * STRIPE + z * ZROWS, ZROWS)],
                sem0).wait()

        plsc.subcore_barrier()

        def src_at(j):
            return src_v.at[j]

        def dst_at(j):
            return dst_v.at[j]

        def make_body(nb):
            def body(i, carry):
                j0 = 2 * i
                pltpu.make_async_copy(
                    h_hbm.at[src_at(j0)], buf0, sem0).wait()
                pltpu.sync_copy(buf0, accum.at[dst_at(j0)], add=True)

                @pl.when(j0 + 2 < nb)
                def _():
                    pltpu.async_copy(h_hbm.at[src_at(j0 + 2)], buf0, sem0)

                j1 = j0 + 1
                pltpu.make_async_copy(
                    h_hbm.at[src_at(j1)], buf1, sem1).wait()
                pltpu.sync_copy(buf1, accum.at[dst_at(j1)], add=True)

                @pl.when(j1 + 2 < nb)
                def _():
                    pltpu.async_copy(h_hbm.at[src_at(j1 + 2)], buf1, sem1)

                return carry
            return body

        pltpu.sync_copy(eidx_hbm.at[pl.ds(start, PHASE_A)], src_v)
        pltpu.sync_copy(eidx_hbm.at[pl.ds(NCHUNKS + start, PHASE_A)], dst_v)
        pltpu.async_copy(h_hbm.at[src_at(0)], buf0, sem0)
        pltpu.async_copy(h_hbm.at[src_at(1)], buf1, sem1)
        lax.fori_loop(0, PHASE_A // 2, make_body(PHASE_A), 0)

        eb = start + PHASE_A
        nbm1 = PHASE_B_MAX - 1
        pltpu.sync_copy(eidx_hbm.at[pl.ds(eb, nbm1)],
                        src_v.at[pl.ds(0, nbm1)])
        pltpu.sync_copy(eidx_hbm.at[pl.ds(NCHUNKS + eb, nbm1)],
                        dst_v.at[pl.ds(0, nbm1)])

        @pl.when(has_extra)
        def _():
            pltpu.sync_copy(eidx_hbm.at[pl.ds(eb + nbm1, 1)],
                            src_v.at[pl.ds(nbm1, 1)])
            pltpu.sync_copy(eidx_hbm.at[pl.ds(NCHUNKS + eb + nbm1, 1)],
                            dst_v.at[pl.ds(nbm1, 1)])

        nb = jnp.where(has_extra, PHASE_B_MAX, PHASE_B_MAX - 1)
        pltpu.async_copy(h_hbm.at[src_at(0)], buf0, sem0)
        pltpu.async_copy(h_hbm.at[src_at(1)], buf1, sem1)
        lax.fori_loop(0, (PHASE_B_MAX - 1) // 2, make_body(nb), 0)

        @pl.when(has_extra)
        def _():
            j = PHASE_B_MAX - 1
            pltpu.make_async_copy(h_hbm.at[src_at(j)], buf0, sem0).wait()
            pltpu.sync_copy(buf0, accum.at[dst_at(j)], add=True)

        plsc.subcore_barrier()

        pltpu.sync_copy(
            accum.at[pl.ds(sid * STRIPE, STRIPE)],
            out_hbm.at[cid, pl.ds(sid * STRIPE, STRIPE)])

        @pl.when(sid == NUM_SUBCORES - 1)
        def _():
            pltpu.sync_copy(
                accum.at[pl.ds(15 * STRIPE + STRIPE, LAST_STRIPE - STRIPE)],
                out_hbm.at[cid, pl.ds(15 * STRIPE + STRIPE,
                                      LAST_STRIPE - STRIPE)])

    return seg_sum(eidx, h)


def _bn_relu_body(part_ref, h_in_ref, gamma_ref, beta_ref, out_ref):
    x = part_ref[0] + part_ref[1] + h_in_ref[...]
    mean = jnp.mean(x, axis=0, keepdims=True)
    xc = x - mean
    var = jnp.mean(xc * xc, axis=0, keepdims=True)
    inv = lax.rsqrt(var + EPS)
    y = xc * inv * gamma_ref[...] + beta_ref[...]
    out_ref[...] = jnp.maximum(y, 0.0)


def _tc_bn_relu(partials, h_in, gamma, beta):
    return pl.pallas_call(
        _bn_relu_body,
        out_shape=jax.ShapeDtypeStruct((N, D), jnp.float32),
    )(partials, h_in, gamma.reshape(1, D), beta.reshape(1, D))


def kernel(edge_index, h, h_in, gamma, beta):
    eidx = edge_index.reshape(2 * NCHUNKS, CHUNK)
    partials = _sc_segment_sum(eidx, h)
    return _tc_bn_relu(partials, h_in, gamma, beta)
